# baseline (speedup 1.0000x reference)
import functools

import jax
import jax.numpy as jnp
from jax.experimental import pallas as pl
from jax.experimental.pallas import tpu as pltpu

B, H, S, D, Q = 8, 8, 2048, 128, 1


def _update_body(pos_ref, k_ref, v_ref, nk_ref, nv_ref, ko_ref, vo_ref):
    b = pl.program_id(0)
    p = pos_ref[b]
    ko_ref[...] = k_ref[...]
    vo_ref[...] = v_ref[...]
    ko_ref[0, 0, p, :] = nk_ref[0, 0, 0, :]
    vo_ref[0, 0, p, :] = nv_ref[0, 0, 0, :]


@jax.jit
def kernel(k_cache, v_cache, new_k, new_v, position_ids):
    pos = position_ids.reshape(B)

    cache_spec = pl.BlockSpec((1, 1, S, D), lambda b, h, pos_ref: (b, h, 0, 0))
    new_spec = pl.BlockSpec((1, 1, Q, D), lambda b, h, pos_ref: (b, h, 0, 0))

    grid_spec = pltpu.PrefetchScalarGridSpec(
        num_scalar_prefetch=1,
        grid=(B, H),
        in_specs=[cache_spec, cache_spec, new_spec, new_spec],
        out_specs=[cache_spec, cache_spec],
    )

    k_out, v_out = pl.pallas_call(
        _update_body,
        grid_spec=grid_spec,
        out_shape=[
            jax.ShapeDtypeStruct((B, H, S, D), k_cache.dtype),
            jax.ShapeDtypeStruct((B, H, S, D), v_cache.dtype),
        ],
    )(pos, k_cache, v_cache, new_k, new_v)
    return (k_out, v_out)
